# baseline (speedup 1.0000x reference)
import jax
import jax.numpy as jnp
from jax import lax
from jax.experimental import pallas as pl
from jax.experimental.pallas import tpu as pltpu
from jax.experimental.pallas import tpu_sc as plsc

B = 16384
K = 32
NC, NS = 2, 16
NW = NC * NS
BPW = B // NW


def _gather_body(u_hbm, m_hbm, ut_hbm, mt_hbm, ue_hbm, me_hbm,
                 uidx_v, midx_v, urows_v, mrows_v, sem_u, sem_m):
    wid = lax.axis_index("s") * NC + lax.axis_index("c")
    base = wid * BPW
    pltpu.sync_copy(u_hbm.at[pl.ds(base, BPW)], uidx_v)
    pltpu.sync_copy(m_hbm.at[pl.ds(base, BPW)], midx_v)
    cu = pltpu.async_copy(ut_hbm.at[uidx_v], urows_v, sem_u)
    cm = pltpu.async_copy(mt_hbm.at[midx_v], mrows_v, sem_m)
    cu.wait()
    cm.wait()
    pltpu.sync_copy(urows_v, ue_hbm.at[pl.ds(base, BPW)])
    pltpu.sync_copy(mrows_v, me_hbm.at[pl.ds(base, BPW)])


_sc_gather = pl.kernel(
    _gather_body,
    out_type=(jax.ShapeDtypeStruct((B, K), jnp.float32),
              jax.ShapeDtypeStruct((B, K), jnp.float32)),
    mesh=plsc.VectorSubcoreMesh(core_axis_name="c", subcore_axis_name="s"),
    scratch_types=[
        pltpu.VMEM((BPW,), jnp.int32),
        pltpu.VMEM((BPW,), jnp.int32),
        pltpu.VMEM((BPW, K), jnp.float32),
        pltpu.VMEM((BPW, K), jnp.float32),
        pltpu.SemaphoreType.DMA,
        pltpu.SemaphoreType.DMA,
    ],
    compiler_params=pltpu.CompilerParams(use_tc_tiling_on_sc=False),
)


BB = 1024


def _mlp_body(ue_ref, me_ref, w1u_ref, w1m_ref, b1_ref, w2_ref, b2_ref,
              w3_ref, b3_ref, out_ref):
    h = jnp.dot(ue_ref[...], w1u_ref[...], preferred_element_type=jnp.float32)
    h = h + jnp.dot(me_ref[...], w1m_ref[...],
                    preferred_element_type=jnp.float32)
    h = jnp.maximum(h + b1_ref[...], 0.0)
    h = jnp.maximum(
        jnp.dot(h, w2_ref[...], preferred_element_type=jnp.float32)
        + b2_ref[...], 0.0)
    out_ref[...] = (
        jnp.dot(h, w3_ref[...], preferred_element_type=jnp.float32)
        + b3_ref[...])


def _mlp(ue, me, w1u, w1m, b1, w2, b2, w3, b3):
    const = lambda i: (0, 0)
    return pl.pallas_call(
        _mlp_body,
        grid=(B // BB,),
        in_specs=[
            pl.BlockSpec((BB, K), lambda i: (i, 0)),
            pl.BlockSpec((BB, K), lambda i: (i, 0)),
            pl.BlockSpec((K, 512), const),
            pl.BlockSpec((K, 512), const),
            pl.BlockSpec((1, 512), const),
            pl.BlockSpec((512, 1024), const),
            pl.BlockSpec((1, 1024), const),
            pl.BlockSpec((1024, 1), const),
            pl.BlockSpec((1, 1), const),
        ],
        out_specs=pl.BlockSpec((BB, 1), lambda i: (i, 0)),
        out_shape=jax.ShapeDtypeStruct((B, 1), jnp.float32),
        compiler_params=pltpu.CompilerParams(
            dimension_semantics=("parallel",)),
    )(ue, me, w1u, w1m, b1, w2, b2, w3, b3)


def kernel(u, m, uTable, mTable, W1, b1, W2, b2, W3, b3):
    u32 = u.reshape(B).astype(jnp.int32)
    m32 = m.reshape(B).astype(jnp.int32)
    ue, me = _sc_gather(u32, m32, uTable, mTable)
    return _mlp(ue, me, W1[:K], W1[K:], b1.reshape(1, 512),
                W2, b2.reshape(1, 1024), W3, b3.reshape(1, 1))


# speedup vs baseline: 3.0672x; 3.0672x over previous
import jax
import jax.numpy as jnp
from jax import lax
from jax.experimental import pallas as pl
from jax.experimental.pallas import tpu as pltpu
from jax.experimental.pallas import tpu_sc as plsc

B = 16384
K = 32
NC, NS = 2, 16
NW = NC * NS
CPW = B // (NW // 2)
G = 8
NG = CPW // G


def _do_gather(idx_hbm, tabT_hbm, out_hbm, w, idx_v, obuf, slabs, sems):
    base = w * CPW
    pltpu.sync_copy(idx_hbm.at[pl.ds(base, CPW)], idx_v.at[pl.ds(0, CPW)])
    lanes = lax.iota(jnp.int32, 16)

    def body(g, carry):
        iv = idx_v[pl.ds(g * G, 16)]
        scal = [jnp.sum(jnp.where(lanes == r, iv, 0)) for r in range(G)]

        @pl.when(g < NG)
        def _fire():
            bank = (g % 2) * G
            for r in range(G):
                col0 = pl.multiple_of((scal[r] >> 7) * 128, 128)
                pltpu.async_copy(tabT_hbm.at[:, pl.ds(col0, 128)],
                                 slabs.at[bank + r], sems.at[g % 2])

        @pl.when(g > 0)
        def _drain():
            bank = ((g - 1) % 2) * G
            for r in range(G):
                pltpu.make_async_copy(tabT_hbm.at[:, pl.ds(0, 128)],
                                      slabs.at[bank + r],
                                      sems.at[(g - 1) % 2]).wait()
            for r in range(G):
                cl = jnp.full((16,), carry[r] & 127, jnp.int32)
                sl = slabs.at[bank + r]
                i = (g - 1) * G + r
                obuf[pl.ds(i * K, 16)] = plsc.load_gather(sl, [lanes, cl])
                obuf[pl.ds(i * K + 16, 16)] = plsc.load_gather(
                    sl, [lanes + 16, cl])

        return tuple(scal)

    lax.fori_loop(0, NG + 1, body, (jnp.int32(0),) * G)
    pltpu.sync_copy(obuf, out_hbm.at[pl.ds(base * K, CPW * K)])


def _gather_body(uidx_hbm, midx_hbm, utT_hbm, mtT_hbm, ue_hbm, me_hbm,
                 idx_v, obuf, slabs, sems):
    wid = lax.axis_index("s") * NC + lax.axis_index("c")

    @pl.when(wid < NW // 2)
    def _():
        _do_gather(uidx_hbm, utT_hbm, ue_hbm, wid, idx_v, obuf, slabs, sems)

    @pl.when(wid >= NW // 2)
    def _():
        _do_gather(midx_hbm, mtT_hbm, me_hbm, wid - NW // 2,
                   idx_v, obuf, slabs, sems)


_sc_gather = pl.kernel(
    _gather_body,
    out_type=(jax.ShapeDtypeStruct((B * K,), jnp.float32),
              jax.ShapeDtypeStruct((B * K,), jnp.float32)),
    mesh=plsc.VectorSubcoreMesh(core_axis_name="c", subcore_axis_name="s"),
    scratch_types=[
        pltpu.VMEM((CPW + 16,), jnp.int32),
        pltpu.VMEM((CPW * K,), jnp.float32),
        pltpu.VMEM((2 * G, K, 128), jnp.float32),
        pltpu.SemaphoreType.DMA((2,)),
    ],
    compiler_params=pltpu.CompilerParams(
        use_tc_tiling_on_sc=True, needs_layout_passes=False),
)


BB = 1024


def _mlp_body(ue_ref, me_ref, w1u_ref, w1m_ref, b1_ref, w2_ref, b2_ref,
              w3_ref, b3_ref, out_ref):
    ue = ue_ref[...].astype(jnp.bfloat16)
    me = me_ref[...].astype(jnp.bfloat16)
    h = jnp.dot(ue, w1u_ref[...], preferred_element_type=jnp.float32)
    h = h + jnp.dot(me, w1m_ref[...], preferred_element_type=jnp.float32)
    h = jnp.maximum(h + b1_ref[...], 0.0).astype(jnp.bfloat16)
    h = jnp.maximum(
        jnp.dot(h, w2_ref[...], preferred_element_type=jnp.float32)
        + b2_ref[...], 0.0).astype(jnp.bfloat16)
    out_ref[...] = (
        jnp.dot(h, w3_ref[...], preferred_element_type=jnp.float32)
        + b3_ref[...])


def _mlp(ue, me, w1u, w1m, b1, w2, b2, w3, b3):
    const = lambda i: (0, 0)
    return pl.pallas_call(
        _mlp_body,
        grid=(B // BB,),
        in_specs=[
            pl.BlockSpec((BB, K), lambda i: (i, 0)),
            pl.BlockSpec((BB, K), lambda i: (i, 0)),
            pl.BlockSpec((K, 512), const),
            pl.BlockSpec((K, 512), const),
            pl.BlockSpec((1, 512), const),
            pl.BlockSpec((512, 1024), const),
            pl.BlockSpec((1, 1024), const),
            pl.BlockSpec((1024, 1), const),
            pl.BlockSpec((1, 1), const),
        ],
        out_specs=pl.BlockSpec((BB, 1), lambda i: (i, 0)),
        out_shape=jax.ShapeDtypeStruct((B, 1), jnp.float32),
        compiler_params=pltpu.CompilerParams(
            dimension_semantics=("parallel",)),
    )(ue, me, w1u, w1m, b1, w2, b2, w3, b3)


def kernel(u, m, uTable, mTable, W1, b1, W2, b2, W3, b3):
    u32 = u.reshape(B).astype(jnp.int32)
    m32 = m.reshape(B).astype(jnp.int32)
    uef, mef = _sc_gather(u32, m32, uTable.T, mTable.T)
    ue, me = uef.reshape(B, K), mef.reshape(B, K)
    w1 = W1.astype(jnp.bfloat16)
    return _mlp(ue, me, w1[:K], w1[K:], b1.reshape(1, 512),
                W2.astype(jnp.bfloat16), b2.reshape(1, 1024),
                W3.astype(jnp.bfloat16), b3.reshape(1, 1))


# speedup vs baseline: 3.2733x; 1.0672x over previous
import jax
import jax.numpy as jnp
from jax import lax
from jax.experimental import pallas as pl
from jax.experimental.pallas import tpu as pltpu
from jax.experimental.pallas import tpu_sc as plsc

B = 16384
K = 32
NC, NS = 2, 16
NW = NC * NS
CPW = B // (NW // 2)
G = 8
NG = CPW // G


HALF = CPW // 2


def _do_gather(idx_hbm, tabT_hbm, out_hbm, w, idx_v, obuf, slabs, sems):
    base = w * CPW
    pltpu.sync_copy(idx_hbm.at[pl.ds(base, CPW)], idx_v.at[pl.ds(0, CPW)])
    lanes = lax.iota(jnp.int32, 16)

    def body(g, carry):
        s1, s2 = carry
        iv = idx_v[pl.ds(g * G, 16)]
        scal = tuple(jnp.sum(jnp.where(lanes == r, iv, 0)) for r in range(G))

        @pl.when(g < NG)
        def _fire():
            bank = (g % 3) * G
            for r in range(G):
                col0 = pl.multiple_of((scal[r] >> 7) * 128, 128)
                pltpu.async_copy(tabT_hbm.at[:, pl.ds(col0, 128)],
                                 slabs.at[bank + r], sems.at[g % 3])

        @pl.when(g == NG // 2 + 2)
        def _flush1():
            pltpu.sync_copy(obuf, out_hbm.at[pl.ds(base * K, HALF * K)])

        @pl.when(g >= 2)
        def _drain():
            bank = ((g - 2) % 3) * G
            for r in range(G):
                pltpu.make_async_copy(tabT_hbm.at[:, pl.ds(0, 128)],
                                      slabs.at[bank + r],
                                      sems.at[(g - 2) % 3]).wait()
            for r in range(G):
                cl = jnp.full((16,), s2[r] & 127, jnp.int32)
                sl = slabs.at[bank + r]
                i = ((g - 2) * G + r) % HALF
                obuf[pl.ds(i * K, 16)] = plsc.load_gather(sl, [lanes, cl])
                obuf[pl.ds(i * K + 16, 16)] = plsc.load_gather(
                    sl, [lanes + 16, cl])

        return (scal, s1)

    z = (jnp.int32(0),) * G
    lax.fori_loop(0, NG + 2, body, (z, z))
    pltpu.sync_copy(obuf, out_hbm.at[pl.ds(base * K + HALF * K, HALF * K)])


def _gather_body(uidx_hbm, midx_hbm, utT_hbm, mtT_hbm, ue_hbm, me_hbm,
                 idx_v, obuf, slabs, sems):
    wid = lax.axis_index("s") * NC + lax.axis_index("c")

    @pl.when(wid < NW // 2)
    def _():
        _do_gather(uidx_hbm, utT_hbm, ue_hbm, wid, idx_v, obuf, slabs, sems)

    @pl.when(wid >= NW // 2)
    def _():
        _do_gather(midx_hbm, mtT_hbm, me_hbm, wid - NW // 2,
                   idx_v, obuf, slabs, sems)


_sc_gather = pl.kernel(
    _gather_body,
    out_type=(jax.ShapeDtypeStruct((B * K,), jnp.float32),
              jax.ShapeDtypeStruct((B * K,), jnp.float32)),
    mesh=plsc.VectorSubcoreMesh(core_axis_name="c", subcore_axis_name="s"),
    scratch_types=[
        pltpu.VMEM((CPW + 32,), jnp.int32),
        pltpu.VMEM((HALF * K,), jnp.float32),
        pltpu.VMEM((3 * G, K, 128), jnp.float32),
        pltpu.SemaphoreType.DMA((3,)),
    ],
    compiler_params=pltpu.CompilerParams(
        use_tc_tiling_on_sc=True, needs_layout_passes=False),
)


BB = 1024


def _mlp_body(ue_ref, me_ref, w1u_ref, w1m_ref, b1_ref, w2_ref, b2_ref,
              w3_ref, b3_ref, out_ref):
    ue = ue_ref[...].astype(jnp.bfloat16)
    me = me_ref[...].astype(jnp.bfloat16)
    h = jnp.dot(ue, w1u_ref[...], preferred_element_type=jnp.float32)
    h = h + jnp.dot(me, w1m_ref[...], preferred_element_type=jnp.float32)
    h = jnp.maximum(h + b1_ref[...], 0.0).astype(jnp.bfloat16)
    h = jnp.maximum(
        jnp.dot(h, w2_ref[...], preferred_element_type=jnp.float32)
        + b2_ref[...], 0.0).astype(jnp.bfloat16)
    out_ref[...] = (
        jnp.dot(h, w3_ref[...], preferred_element_type=jnp.float32)
        + b3_ref[...])


def _mlp(ue, me, w1u, w1m, b1, w2, b2, w3, b3):
    const = lambda i: (0, 0)
    return pl.pallas_call(
        _mlp_body,
        grid=(B // BB,),
        in_specs=[
            pl.BlockSpec((BB, K), lambda i: (i, 0)),
            pl.BlockSpec((BB, K), lambda i: (i, 0)),
            pl.BlockSpec((K, 512), const),
            pl.BlockSpec((K, 512), const),
            pl.BlockSpec((1, 512), const),
            pl.BlockSpec((512, 1024), const),
            pl.BlockSpec((1, 1024), const),
            pl.BlockSpec((1024, 1), const),
            pl.BlockSpec((1, 1), const),
        ],
        out_specs=pl.BlockSpec((BB, 1), lambda i: (i, 0)),
        out_shape=jax.ShapeDtypeStruct((B, 1), jnp.float32),
        compiler_params=pltpu.CompilerParams(
            dimension_semantics=("parallel",)),
    )(ue, me, w1u, w1m, b1, w2, b2, w3, b3)


def kernel(u, m, uTable, mTable, W1, b1, W2, b2, W3, b3):
    u32 = u.reshape(B).astype(jnp.int32)
    m32 = m.reshape(B).astype(jnp.int32)
    uef, mef = _sc_gather(u32, m32, uTable.T, mTable.T)
    ue, me = uef.reshape(B, K), mef.reshape(B, K)
    w1 = W1.astype(jnp.bfloat16)
    return _mlp(ue, me, w1[:K], w1[K:], b1.reshape(1, 512),
                W2.astype(jnp.bfloat16), b2.reshape(1, 1024),
                W3.astype(jnp.bfloat16), b3.reshape(1, 1))


# speedup vs baseline: 3.5197x; 1.0753x over previous
import jax
import jax.numpy as jnp
from jax import lax
from jax.experimental import pallas as pl
from jax.experimental.pallas import tpu as pltpu
from jax.experimental.pallas import tpu_sc as plsc

B = 16384
K = 32
NBC = 2
BC = B // NBC
NC, NS = 2, 16
NW = NC * NS
CPW = BC // (NW // 2)
G = 8
NG = CPW // G


HALF = CPW // 2


def _do_gather(idx_hbm, tabT_hbm, out_hbm, w, idx_v, obuf, slabs, sems):
    base = w * CPW
    pltpu.sync_copy(idx_hbm.at[pl.ds(base, CPW)], idx_v.at[pl.ds(0, CPW)])
    lanes = lax.iota(jnp.int32, 16)

    def body(g, carry):
        s1, s2 = carry
        iv = idx_v[pl.ds(g * G, 16)]
        scal = tuple(jnp.sum(jnp.where(lanes == r, iv, 0)) for r in range(G))

        @pl.when(g < NG)
        def _fire():
            bank = (g % 3) * G
            for r in range(G):
                col0 = pl.multiple_of((scal[r] >> 7) * 128, 128)
                pltpu.async_copy(tabT_hbm.at[:, pl.ds(col0, 128)],
                                 slabs.at[bank + r], sems.at[g % 3])

        @pl.when(g == NG // 2 + 2)
        def _flush1():
            pltpu.sync_copy(obuf, out_hbm.at[pl.ds(base * K, HALF * K)])

        @pl.when(g >= 2)
        def _drain():
            bank = ((g - 2) % 3) * G
            for r in range(G):
                pltpu.make_async_copy(tabT_hbm.at[:, pl.ds(0, 128)],
                                      slabs.at[bank + r],
                                      sems.at[(g - 2) % 3]).wait()
            for r in range(G):
                cl = jnp.full((16,), s2[r] & 127, jnp.int32)
                sl = slabs.at[bank + r]
                i = ((g - 2) * G + r) % HALF
                obuf[pl.ds(i * K, 16)] = plsc.load_gather(sl, [lanes, cl])
                obuf[pl.ds(i * K + 16, 16)] = plsc.load_gather(
                    sl, [lanes + 16, cl])

        return (scal, s1)

    z = (jnp.int32(0),) * G
    lax.fori_loop(0, NG + 2, body, (z, z))
    pltpu.sync_copy(obuf, out_hbm.at[pl.ds(base * K + HALF * K, HALF * K)])


def _gather_body(uidx_hbm, midx_hbm, utT_hbm, mtT_hbm, ue_hbm, me_hbm,
                 idx_v, obuf, slabs, sems):
    wid = lax.axis_index("s") * NC + lax.axis_index("c")

    @pl.when(wid < NW // 2)
    def _():
        _do_gather(uidx_hbm, utT_hbm, ue_hbm, wid, idx_v, obuf, slabs, sems)

    @pl.when(wid >= NW // 2)
    def _():
        _do_gather(midx_hbm, mtT_hbm, me_hbm, wid - NW // 2,
                   idx_v, obuf, slabs, sems)


_sc_gather = pl.kernel(
    _gather_body,
    out_type=(jax.ShapeDtypeStruct((BC * K,), jnp.float32),
              jax.ShapeDtypeStruct((BC * K,), jnp.float32)),
    mesh=plsc.VectorSubcoreMesh(core_axis_name="c", subcore_axis_name="s"),
    scratch_types=[
        pltpu.VMEM((CPW + 32,), jnp.int32),
        pltpu.VMEM((HALF * K,), jnp.float32),
        pltpu.VMEM((3 * G, K, 128), jnp.float32),
        pltpu.SemaphoreType.DMA((3,)),
    ],
    compiler_params=pltpu.CompilerParams(
        use_tc_tiling_on_sc=True, needs_layout_passes=False),
)


BB = 1024


def _mlp_body(ue_ref, me_ref, w1u_ref, w1m_ref, b1_ref, w2_ref, b2_ref,
              w3_ref, b3_ref, out_ref):
    ue = ue_ref[...].astype(jnp.bfloat16)
    me = me_ref[...].astype(jnp.bfloat16)
    h = jnp.dot(ue, w1u_ref[...], preferred_element_type=jnp.float32)
    h = h + jnp.dot(me, w1m_ref[...], preferred_element_type=jnp.float32)
    h = jnp.maximum(h + b1_ref[...], 0.0).astype(jnp.bfloat16)
    h = jnp.maximum(
        jnp.dot(h, w2_ref[...], preferred_element_type=jnp.float32)
        + b2_ref[...], 0.0).astype(jnp.bfloat16)
    out_ref[...] = (
        jnp.dot(h, w3_ref[...], preferred_element_type=jnp.float32)
        + b3_ref[...])


def _mlp(ue, me, w1u, w1m, b1, w2, b2, w3, b3):
    const = lambda i: (0, 0)
    return pl.pallas_call(
        _mlp_body,
        grid=(BC // BB,),
        in_specs=[
            pl.BlockSpec((BB, K), lambda i: (i, 0)),
            pl.BlockSpec((BB, K), lambda i: (i, 0)),
            pl.BlockSpec((K, 512), const),
            pl.BlockSpec((K, 512), const),
            pl.BlockSpec((1, 512), const),
            pl.BlockSpec((512, 1024), const),
            pl.BlockSpec((1, 1024), const),
            pl.BlockSpec((1024, 1), const),
            pl.BlockSpec((1, 1), const),
        ],
        out_specs=pl.BlockSpec((BB, 1), lambda i: (i, 0)),
        out_shape=jax.ShapeDtypeStruct((BC, 1), jnp.float32),
        compiler_params=pltpu.CompilerParams(
            dimension_semantics=("parallel",)),
    )(ue, me, w1u, w1m, b1, w2, b2, w3, b3)


def kernel(u, m, uTable, mTable, W1, b1, W2, b2, W3, b3):
    u32 = u.reshape(B).astype(jnp.int32)
    m32 = m.reshape(B).astype(jnp.int32)
    utT, mtT = uTable.T, mTable.T
    w1 = W1.astype(jnp.bfloat16)
    w2 = W2.astype(jnp.bfloat16)
    w3 = W3.astype(jnp.bfloat16)
    b1r, b2r, b3r = b1.reshape(1, 512), b2.reshape(1, 1024), b3.reshape(1, 1)
    outs = []
    for c in range(NBC):
        uef, mef = _sc_gather(u32[c * BC:(c + 1) * BC],
                              m32[c * BC:(c + 1) * BC], utT, mtT)
        outs.append(_mlp(uef.reshape(BC, K), mef.reshape(BC, K),
                         w1[:K], w1[K:], b1r, w2, b2r, w3, b3r))
    return jnp.concatenate(outs, axis=0)


# speedup vs baseline: 3.5287x; 1.0026x over previous
import jax
import jax.numpy as jnp
from jax import lax
from jax.experimental import pallas as pl
from jax.experimental.pallas import tpu as pltpu
from jax.experimental.pallas import tpu_sc as plsc

B = 16384
K = 32
NBC = 4
BC = B // NBC
NC, NS = 2, 16
NW = NC * NS
CPW = BC // (NW // 2)
G = 8
NG = CPW // G


HALF = CPW // 2


def _do_gather(idx_hbm, tabT_hbm, out_hbm, w, idx_v, obuf, slabs, sems):
    base = w * CPW
    pltpu.sync_copy(idx_hbm.at[pl.ds(base, CPW)], idx_v.at[pl.ds(0, CPW)])
    lanes = lax.iota(jnp.int32, 16)

    def body(g, carry):
        s1, s2 = carry
        iv = idx_v[pl.ds(g * G, 16)]
        scal = tuple(jnp.sum(jnp.where(lanes == r, iv, 0)) for r in range(G))

        @pl.when(g < NG)
        def _fire():
            bank = (g % 3) * G
            for r in range(G):
                col0 = pl.multiple_of((scal[r] >> 7) * 128, 128)
                pltpu.async_copy(tabT_hbm.at[:, pl.ds(col0, 128)],
                                 slabs.at[bank + r], sems.at[g % 3])

        @pl.when(g == NG // 2 + 2)
        def _flush1():
            pltpu.sync_copy(obuf, out_hbm.at[pl.ds(base * K, HALF * K)])

        @pl.when(g >= 2)
        def _drain():
            bank = ((g - 2) % 3) * G
            for r in range(G):
                pltpu.make_async_copy(tabT_hbm.at[:, pl.ds(0, 128)],
                                      slabs.at[bank + r],
                                      sems.at[(g - 2) % 3]).wait()
            for r in range(G):
                cl = jnp.full((16,), s2[r] & 127, jnp.int32)
                sl = slabs.at[bank + r]
                i = ((g - 2) * G + r) % HALF
                obuf[pl.ds(i * K, 16)] = plsc.load_gather(sl, [lanes, cl])
                obuf[pl.ds(i * K + 16, 16)] = plsc.load_gather(
                    sl, [lanes + 16, cl])

        return (scal, s1)

    z = (jnp.int32(0),) * G
    lax.fori_loop(0, NG + 2, body, (z, z))
    pltpu.sync_copy(obuf, out_hbm.at[pl.ds(base * K + HALF * K, HALF * K)])


def _gather_body(uidx_hbm, midx_hbm, utT_hbm, mtT_hbm, ue_hbm, me_hbm,
                 idx_v, obuf, slabs, sems):
    wid = lax.axis_index("s") * NC + lax.axis_index("c")

    @pl.when(wid < NW // 2)
    def _():
        _do_gather(uidx_hbm, utT_hbm, ue_hbm, wid, idx_v, obuf, slabs, sems)

    @pl.when(wid >= NW // 2)
    def _():
        _do_gather(midx_hbm, mtT_hbm, me_hbm, wid - NW // 2,
                   idx_v, obuf, slabs, sems)


_sc_gather = pl.kernel(
    _gather_body,
    out_type=(jax.ShapeDtypeStruct((BC * K,), jnp.float32),
              jax.ShapeDtypeStruct((BC * K,), jnp.float32)),
    mesh=plsc.VectorSubcoreMesh(core_axis_name="c", subcore_axis_name="s"),
    scratch_types=[
        pltpu.VMEM((CPW + 32,), jnp.int32),
        pltpu.VMEM((HALF * K,), jnp.float32),
        pltpu.VMEM((3 * G, K, 128), jnp.float32),
        pltpu.SemaphoreType.DMA((3,)),
    ],
    compiler_params=pltpu.CompilerParams(
        use_tc_tiling_on_sc=True, needs_layout_passes=False),
)


BB = 1024


def _mlp_body(ue_ref, me_ref, w1u_ref, w1m_ref, b1_ref, w2_ref, b2_ref,
              w3_ref, b3_ref, out_ref):
    ue = ue_ref[...].astype(jnp.bfloat16)
    me = me_ref[...].astype(jnp.bfloat16)
    h = jnp.dot(ue, w1u_ref[...], preferred_element_type=jnp.float32)
    h = h + jnp.dot(me, w1m_ref[...], preferred_element_type=jnp.float32)
    h = jnp.maximum(h + b1_ref[...], 0.0).astype(jnp.bfloat16)
    h = jnp.maximum(
        jnp.dot(h, w2_ref[...], preferred_element_type=jnp.float32)
        + b2_ref[...], 0.0).astype(jnp.bfloat16)
    out_ref[...] = (
        jnp.dot(h, w3_ref[...], preferred_element_type=jnp.float32)
        + b3_ref[...])


def _mlp(ue, me, w1u, w1m, b1, w2, b2, w3, b3):
    const = lambda i: (0, 0)
    return pl.pallas_call(
        _mlp_body,
        grid=(BC // BB,),
        in_specs=[
            pl.BlockSpec((BB, K), lambda i: (i, 0)),
            pl.BlockSpec((BB, K), lambda i: (i, 0)),
            pl.BlockSpec((K, 512), const),
            pl.BlockSpec((K, 512), const),
            pl.BlockSpec((1, 512), const),
            pl.BlockSpec((512, 1024), const),
            pl.BlockSpec((1, 1024), const),
            pl.BlockSpec((1024, 1), const),
            pl.BlockSpec((1, 1), const),
        ],
        out_specs=pl.BlockSpec((BB, 1), lambda i: (i, 0)),
        out_shape=jax.ShapeDtypeStruct((BC, 1), jnp.float32),
        compiler_params=pltpu.CompilerParams(
            dimension_semantics=("parallel",)),
    )(ue, me, w1u, w1m, b1, w2, b2, w3, b3)


def kernel(u, m, uTable, mTable, W1, b1, W2, b2, W3, b3):
    u32 = u.reshape(B).astype(jnp.int32)
    m32 = m.reshape(B).astype(jnp.int32)
    utT, mtT = uTable.T, mTable.T
    w1 = W1.astype(jnp.bfloat16)
    w2 = W2.astype(jnp.bfloat16)
    w3 = W3.astype(jnp.bfloat16)
    b1r, b2r, b3r = b1.reshape(1, 512), b2.reshape(1, 1024), b3.reshape(1, 1)
    outs = []
    for c in range(NBC):
        uef, mef = _sc_gather(u32[c * BC:(c + 1) * BC],
                              m32[c * BC:(c + 1) * BC], utT, mtT)
        outs.append(_mlp(uef.reshape(BC, K), mef.reshape(BC, K),
                         w1[:K], w1[K:], b1r, w2, b2r, w3, b3r))
    return jnp.concatenate(outs, axis=0)


# speedup vs baseline: 3.5909x; 1.0176x over previous
import jax
import jax.numpy as jnp
from jax import lax
from jax.experimental import pallas as pl
from jax.experimental.pallas import tpu as pltpu
from jax.experimental.pallas import tpu_sc as plsc

B = 16384
K = 32
NBC = 4
BC = B // NBC
NC, NS = 2, 16
NW = NC * NS
CPW = BC // (NW // 2)
G = 8
NG = CPW // G


HALF = CPW // 2


def _do_gather(idx_hbm, tabT_hbm, out_hbm, w, idx_v, obuf, slabs, sems):
    base = w * CPW
    pltpu.sync_copy(idx_hbm.at[pl.ds(base, CPW)], idx_v.at[pl.ds(0, CPW)])
    lanes = lax.iota(jnp.int32, 16)

    def body(g, carry):
        s1, s2 = carry
        iv = idx_v[pl.ds(g * G, 16)]
        scal = tuple(jnp.sum(jnp.where(lanes == r, iv, 0)) for r in range(G))

        @pl.when(g < NG)
        def _fire():
            bank = (g % 3) * G
            for r in range(G):
                col0 = pl.multiple_of((scal[r] >> 7) * 128, 128)
                for q in range(K // 8):
                    pltpu.async_copy(
                        tabT_hbm.at[pl.ds(q * 8, 8), pl.ds(col0, 128)],
                        slabs.at[bank + r, pl.ds(q * 8, 8)],
                        sems.at[g % 3])

        @pl.when(g == NG // 2 + 2)
        def _flush1():
            pltpu.sync_copy(obuf, out_hbm.at[pl.ds(base * K, HALF * K)])

        @pl.when(g >= 2)
        def _drain():
            bank = ((g - 2) % 3) * G
            for r in range(G):
                pltpu.make_async_copy(tabT_hbm.at[:, pl.ds(0, 128)],
                                      slabs.at[bank + r],
                                      sems.at[(g - 2) % 3]).wait()
            for r in range(G):
                cl = jnp.full((16,), s2[r] & 127, jnp.int32)
                sl = slabs.at[bank + r]
                i = ((g - 2) * G + r) % HALF
                obuf[pl.ds(i * K, 16)] = plsc.load_gather(sl, [lanes, cl])
                obuf[pl.ds(i * K + 16, 16)] = plsc.load_gather(
                    sl, [lanes + 16, cl])

        return (scal, s1)

    z = (jnp.int32(0),) * G
    lax.fori_loop(0, NG + 2, body, (z, z))
    pltpu.sync_copy(obuf, out_hbm.at[pl.ds(base * K + HALF * K, HALF * K)])


def _gather_body(uidx_hbm, midx_hbm, utT_hbm, mtT_hbm, ue_hbm, me_hbm,
                 idx_v, obuf, slabs, sems):
    wid = lax.axis_index("s") * NC + lax.axis_index("c")

    @pl.when(wid < NW // 2)
    def _():
        _do_gather(uidx_hbm, utT_hbm, ue_hbm, wid, idx_v, obuf, slabs, sems)

    @pl.when(wid >= NW // 2)
    def _():
        _do_gather(midx_hbm, mtT_hbm, me_hbm, wid - NW // 2,
                   idx_v, obuf, slabs, sems)


_sc_gather = pl.kernel(
    _gather_body,
    out_type=(jax.ShapeDtypeStruct((BC * K,), jnp.float32),
              jax.ShapeDtypeStruct((BC * K,), jnp.float32)),
    mesh=plsc.VectorSubcoreMesh(core_axis_name="c", subcore_axis_name="s"),
    scratch_types=[
        pltpu.VMEM((CPW + 32,), jnp.int32),
        pltpu.VMEM((HALF * K,), jnp.float32),
        pltpu.VMEM((3 * G, K, 128), jnp.float32),
        pltpu.SemaphoreType.DMA((3,)),
    ],
    compiler_params=pltpu.CompilerParams(
        use_tc_tiling_on_sc=True, needs_layout_passes=False),
)


BB = 1024


def _mlp_body(ue_ref, me_ref, w1u_ref, w1m_ref, b1_ref, w2_ref, b2_ref,
              w3_ref, b3_ref, out_ref):
    ue = ue_ref[...].astype(jnp.bfloat16)
    me = me_ref[...].astype(jnp.bfloat16)
    h = jnp.dot(ue, w1u_ref[...], preferred_element_type=jnp.float32)
    h = h + jnp.dot(me, w1m_ref[...], preferred_element_type=jnp.float32)
    h = jnp.maximum(h + b1_ref[...], 0.0).astype(jnp.bfloat16)
    h = jnp.maximum(
        jnp.dot(h, w2_ref[...], preferred_element_type=jnp.float32)
        + b2_ref[...], 0.0).astype(jnp.bfloat16)
    out_ref[...] = (
        jnp.dot(h, w3_ref[...], preferred_element_type=jnp.float32)
        + b3_ref[...])


def _mlp(ue, me, w1u, w1m, b1, w2, b2, w3, b3):
    const = lambda i: (0, 0)
    return pl.pallas_call(
        _mlp_body,
        grid=(BC // BB,),
        in_specs=[
            pl.BlockSpec((BB, K), lambda i: (i, 0)),
            pl.BlockSpec((BB, K), lambda i: (i, 0)),
            pl.BlockSpec((K, 512), const),
            pl.BlockSpec((K, 512), const),
            pl.BlockSpec((1, 512), const),
            pl.BlockSpec((512, 1024), const),
            pl.BlockSpec((1, 1024), const),
            pl.BlockSpec((1024, 1), const),
            pl.BlockSpec((1, 1), const),
        ],
        out_specs=pl.BlockSpec((BB, 1), lambda i: (i, 0)),
        out_shape=jax.ShapeDtypeStruct((BC, 1), jnp.float32),
        compiler_params=pltpu.CompilerParams(
            dimension_semantics=("parallel",)),
    )(ue, me, w1u, w1m, b1, w2, b2, w3, b3)


def kernel(u, m, uTable, mTable, W1, b1, W2, b2, W3, b3):
    u32 = u.reshape(B).astype(jnp.int32)
    m32 = m.reshape(B).astype(jnp.int32)
    utT, mtT = uTable.T, mTable.T
    w1 = W1.astype(jnp.bfloat16)
    w2 = W2.astype(jnp.bfloat16)
    w3 = W3.astype(jnp.bfloat16)
    b1r, b2r, b3r = b1.reshape(1, 512), b2.reshape(1, 1024), b3.reshape(1, 1)
    outs = []
    for c in range(NBC):
        uef, mef = _sc_gather(u32[c * BC:(c + 1) * BC],
                              m32[c * BC:(c + 1) * BC], utT, mtT)
        outs.append(_mlp(uef.reshape(BC, K), mef.reshape(BC, K),
                         w1[:K], w1[K:], b1r, w2, b2r, w3, b3r))
    return jnp.concatenate(outs, axis=0)
